# baseline (speedup 1.0000x reference)
import functools

import jax
import jax.numpy as jnp
from jax import lax
from jax.experimental import pallas as pl
from jax.experimental.pallas import tpu as pltpu
from jax.experimental.pallas import tpu_sc as plsc

_B, _H = 16384, 200
_HPW = 7
_ROW = _B
_OUT_ROW = 2 * _B


def _sc_kernel(x_hbm, y_hbm, f_hbm, xv0, yv0, xv1, yv1, ov, sem0, sem1):
    nc = 2
    wid = lax.axis_index("s") * nc + lax.axis_index("c")
    scale = jnp.float32(2.0 / 3.0)
    h0 = wid * _HPW
    xbufs, ybufs, sems = (xv0, xv1), (yv0, yv1), (sem0, sem1)

    def start_in(h, slot):
        cx = pltpu.make_async_copy(x_hbm.at[h], xbufs[slot], sems[slot])
        cy = pltpu.make_async_copy(y_hbm.at[h], ybufs[slot], sems[slot])
        cx.start()
        cy.start()
        return cx, cy

    def compute(slot):
        xv, yv = xbufs[slot], ybufs[slot]

        def tb_body(t, carry):
            for u in range(4):
                tb = t * 4 + u
                for j in range(8):
                    b = tb * 128 + j * 16
                    v = xv[pl.ds(b, 16)].astype(jnp.float32) * scale - 1.0
                    ov[pl.ds(tb * 256 + j * 16, 16)] = v
                for j in range(8):
                    b = tb * 128 + j * 16
                    v = yv[pl.ds(b, 16)].astype(jnp.float32) * scale - 1.0
                    ov[pl.ds(tb * 256 + 128 + j * 16, 16)] = v
            return carry

        lax.fori_loop(0, 32, tb_body, 0)

    @pl.when(h0 < _H)
    def _():
        pend = start_in(h0, 0)
        for i in range(_HPW):
            slot = i % 2
            h = h0 + i

            @pl.when(h < _H)
            def _(pend=pend, slot=slot, h=h):
                if i + 1 < _HPW:

                    @pl.when(h + 1 < _H)
                    def _():
                        start_in(h + 1, 1 - slot)

                pend[0].wait()
                pend[1].wait()
                compute(slot)
                pltpu.sync_copy(ov, f_hbm.at[pl.ds(h * _OUT_ROW, _OUT_ROW)])

            if i + 1 < _HPW:
                pend = (
                    pltpu.make_async_copy(x_hbm.at[h0], xbufs[1 - slot], sems[1 - slot]),
                    pltpu.make_async_copy(y_hbm.at[h0], ybufs[1 - slot], sems[1 - slot]),
                )


def _make_sc_call():
    mesh = plsc.VectorSubcoreMesh(core_axis_name="c", subcore_axis_name="s")
    return functools.partial(
        pl.kernel,
        mesh=mesh,
        out_type=jax.ShapeDtypeStruct((_H * _OUT_ROW,), jnp.float32),
        scratch_types=[
            pltpu.VMEM((_ROW,), jnp.int32),
            pltpu.VMEM((_ROW,), jnp.int32),
            pltpu.VMEM((_ROW,), jnp.int32),
            pltpu.VMEM((_ROW,), jnp.int32),
            pltpu.VMEM((_OUT_ROW,), jnp.float32),
            pltpu.SemaphoreType.DMA,
            pltpu.SemaphoreType.DMA,
        ],
    )(_sc_kernel)


def kernel(x_x, x_y):
    xt = x_x.T
    yt = x_y.T
    f1 = _make_sc_call()(xt, yt)
    f = f1.reshape(_H * 256, 128)
    cf = f.astype(jnp.complex64)
    out = cf.reshape(_H, 128, 2, 128).transpose(1, 3, 0, 2)
    return out.reshape(_B, _H, 2)


# speedup vs baseline: 1.0154x; 1.0154x over previous
import functools

import jax
import jax.numpy as jnp
from jax import lax
from jax.experimental import pallas as pl
from jax.experimental.pallas import tpu as pltpu
from jax.experimental.pallas import tpu_sc as plsc

_B, _H = 16384, 200
_HPW = 7
_ROW = _B
_OUT_ROW = 2 * _B


def _sc_kernel(x_hbm, y_hbm, f_hbm, xv0, yv0, xv1, yv1, ov, sem0, sem1):
    nc = 2
    wid = lax.axis_index("s") * nc + lax.axis_index("c")
    scale = jnp.float32(2.0 / 3.0)
    h0 = wid * _HPW
    xbufs, ybufs, sems = (xv0, xv1), (yv0, yv1), (sem0, sem1)

    def start_in(h, slot):
        cx = pltpu.make_async_copy(x_hbm.at[h], xbufs[slot], sems[slot])
        cy = pltpu.make_async_copy(y_hbm.at[h], ybufs[slot], sems[slot])
        cx.start()
        cy.start()
        return cx, cy

    def compute(slot):
        xv, yv = xbufs[slot], ybufs[slot]

        def tb_body(tb, carry):
            for j in range(8):
                b = tb * 128 + j * 16
                v = xv[pl.ds(b, 16)].astype(jnp.float32) * scale - 1.0
                ov[pl.ds(tb * 256 + j * 16, 16)] = v
            for j in range(8):
                b = tb * 128 + j * 16
                v = yv[pl.ds(b, 16)].astype(jnp.float32) * scale - 1.0
                ov[pl.ds(tb * 256 + 128 + j * 16, 16)] = v
            return carry

        lax.fori_loop(0, 128, tb_body, 0)

    @pl.when(h0 < _H)
    def _():
        pend = start_in(h0, 0)
        for i in range(_HPW):
            slot = i % 2
            h = h0 + i

            @pl.when(h < _H)
            def _(pend=pend, slot=slot, h=h):
                if i + 1 < _HPW:

                    @pl.when(h + 1 < _H)
                    def _():
                        start_in(h + 1, 1 - slot)

                pend[0].wait()
                pend[1].wait()
                compute(slot)
                pltpu.sync_copy(ov, f_hbm.at[pl.ds(h * _OUT_ROW, _OUT_ROW)])

            if i + 1 < _HPW:
                pend = (
                    pltpu.make_async_copy(x_hbm.at[h0], xbufs[1 - slot], sems[1 - slot]),
                    pltpu.make_async_copy(y_hbm.at[h0], ybufs[1 - slot], sems[1 - slot]),
                )


def _make_sc_call():
    mesh = plsc.VectorSubcoreMesh(core_axis_name="c", subcore_axis_name="s")
    return functools.partial(
        pl.kernel,
        mesh=mesh,
        out_type=jax.ShapeDtypeStruct((_H * _OUT_ROW,), jnp.float32),
        scratch_types=[
            pltpu.VMEM((_ROW,), jnp.int32),
            pltpu.VMEM((_ROW,), jnp.int32),
            pltpu.VMEM((_ROW,), jnp.int32),
            pltpu.VMEM((_ROW,), jnp.int32),
            pltpu.VMEM((_OUT_ROW,), jnp.float32),
            pltpu.SemaphoreType.DMA,
            pltpu.SemaphoreType.DMA,
        ],
    )(_sc_kernel)


def kernel(x_x, x_y):
    xt = x_x.T
    yt = x_y.T
    f1 = _make_sc_call()(xt, yt)
    f = f1.reshape(_H * 256, 128)
    cf = f.astype(jnp.complex64)
    out = cf.reshape(_H, 128, 2, 128).transpose(1, 3, 0, 2)
    return out.reshape(_B, _H, 2)
